# baseline (speedup 1.0000x reference)
import jax
import jax.numpy as jnp
from jax.experimental import pallas as pl
from jax.experimental.pallas import tpu as pltpu


def _bcast_kernel(tid_ref, src_ref, out_ref):
    del tid_ref
    out_ref[...] = jnp.broadcast_to(src_ref[...], out_ref.shape)


def kernel(x_query, vis_mark, e_p, task_id):
    del vis_mark
    B = x_query.shape[0]
    nL, _, P, D = e_p.shape
    tid = jnp.asarray(task_id, jnp.int32).reshape((1,))
    b_blk = 32
    grid = (nL, B // b_blk)
    return pl.pallas_call(
        _bcast_kernel,
        grid_spec=pltpu.PrefetchScalarGridSpec(
            num_scalar_prefetch=1,
            grid=grid,
            in_specs=[
                pl.BlockSpec((1, 1, P, D), lambda l, b, tid: (l, tid[0], 0, 0)),
            ],
            out_specs=pl.BlockSpec((1, b_blk, P, D), lambda l, b, tid: (l, b, 0, 0)),
        ),
        out_shape=jax.ShapeDtypeStruct((nL, B, P, D), e_p.dtype),
    )(tid, e_p)


# speedup vs baseline: 1.0672x; 1.0672x over previous
import jax
import jax.numpy as jnp
from jax.experimental import pallas as pl
from jax.experimental.pallas import tpu as pltpu


def _bcast_kernel(tid_ref, src_ref, out_ref):
    del tid_ref
    out_ref[...] = jnp.broadcast_to(src_ref[...], out_ref.shape)


def kernel(x_query, vis_mark, e_p, task_id):
    del vis_mark
    B = x_query.shape[0]
    nL, _, P, D = e_p.shape
    tid = jnp.asarray(task_id, jnp.int32).reshape((1,))
    b_blk = 128
    grid = (nL, B // b_blk)
    return pl.pallas_call(
        _bcast_kernel,
        grid_spec=pltpu.PrefetchScalarGridSpec(
            num_scalar_prefetch=1,
            grid=grid,
            in_specs=[
                pl.BlockSpec((1, 1, P, D), lambda l, b, tid: (l, tid[0], 0, 0)),
            ],
            out_specs=pl.BlockSpec((1, b_blk, P, D), lambda l, b, tid: (l, b, 0, 0)),
        ),
        out_shape=jax.ShapeDtypeStruct((nL, B, P, D), e_p.dtype),
    )(tid, e_p)
